# Initial kernel scaffold; baseline (speedup 1.0000x reference)
import jax
import jax.numpy as jnp
from jax.experimental import pallas as pl


def kernel(patch_indices, patch_embeddings):
    raise NotImplementedError("write your pallas kernel here")


# speedup vs baseline: 4.3046x; 4.3046x over previous
import functools

import jax
import jax.numpy as jnp
from jax import lax
from jax.experimental import pallas as pl
from jax.experimental.pallas import tpu as pltpu
from jax.experimental.pallas import tpu_sc as plsc


def _gather_kernel(B, V, D, NW, C):
    n_chunks_per_w = B // (NW * C)
    b_per_w = B // NW
    mesh = plsc.VectorSubcoreMesh(core_axis_name="c", subcore_axis_name="s")

    @functools.partial(
        pl.kernel,
        out_type=jax.ShapeDtypeStruct((B, D), jnp.float32),
        mesh=mesh,
        scratch_types=[
            pltpu.VMEM((C,), jnp.int32),
            pltpu.VMEM((C, D), jnp.float32),
            pltpu.SemaphoreType.DMA,
        ],
        compiler_params=pltpu.CompilerParams(use_tc_tiling_on_sc=False),
    )
    def k(idx_hbm, table_hbm, out_hbm, idx_v, rows_v, sem):
        nc = 2
        wid = lax.axis_index("s") * nc + lax.axis_index("c")
        base = wid * b_per_w

        def body(g, carry):
            off = base + g * C
            pltpu.sync_copy(idx_hbm.at[pl.ds(off, C)], idx_v)
            pltpu.async_copy(table_hbm.at[idx_v], rows_v, sem).wait()
            pltpu.sync_copy(rows_v, out_hbm.at[pl.ds(off, C)])
            return carry

        lax.fori_loop(0, n_chunks_per_w, body, 0)

    return k


def kernel(patch_indices, patch_embeddings):
    B = patch_indices.shape[0]
    V, D = patch_embeddings.shape
    NW = 32
    C = 512
    idx = patch_indices.astype(jnp.int32)
    return _gather_kernel(B, V, D, NW, C)(idx, patch_embeddings)


# speedup vs baseline: 4.5772x; 1.0633x over previous
import functools

import jax
import jax.numpy as jnp
from jax import lax
from jax.experimental import pallas as pl
from jax.experimental.pallas import tpu as pltpu
from jax.experimental.pallas import tpu_sc as plsc

NW = 32
NBUF = 4
CHUNK = 256


def _gather_kernel(B, V, D):
    b_per_w = B // NW
    n_chunks = b_per_w // CHUNK
    n_outer = n_chunks // NBUF
    mesh = plsc.VectorSubcoreMesh(core_axis_name="c", subcore_axis_name="s")

    scratch = (
        [pltpu.VMEM((CHUNK,), jnp.int32) for _ in range(NBUF)]
        + [pltpu.VMEM((CHUNK, D), jnp.float32) for _ in range(NBUF)]
        + [pltpu.SemaphoreType.DMA for _ in range(3 * NBUF)]
    )

    @functools.partial(
        pl.kernel,
        out_type=jax.ShapeDtypeStruct((B, D), jnp.float32),
        mesh=mesh,
        scratch_types=scratch,
        compiler_params=pltpu.CompilerParams(use_tc_tiling_on_sc=False),
    )
    def k(idx_hbm, table_hbm, out_hbm, *scratch_refs):
        idx_bufs = scratch_refs[:NBUF]
        rows_bufs = scratch_refs[NBUF : 2 * NBUF]
        sem_i = scratch_refs[2 * NBUF : 3 * NBUF]
        sem_g = scratch_refs[3 * NBUF : 4 * NBUF]
        sem_o = scratch_refs[4 * NBUF : 5 * NBUF]

        nc = 2
        wid = lax.axis_index("s") * nc + lax.axis_index("c")
        base = wid * b_per_w

        def issue_idx(g, b):
            pltpu.async_copy(
                idx_hbm.at[pl.ds(base + g * CHUNK, CHUNK)], idx_bufs[b], sem_i[b]
            )

        def wait_idx(b):
            pltpu.make_async_copy(
                idx_hbm.at[pl.ds(0, CHUNK)], idx_bufs[b], sem_i[b]
            ).wait()

        def issue_gather(b):
            pltpu.async_copy(table_hbm.at[idx_bufs[b]], rows_bufs[b], sem_g[b])

        def wait_gather(b):
            pltpu.make_async_copy(
                table_hbm.at[idx_bufs[b]], rows_bufs[b], sem_g[b]
            ).wait()

        def issue_out(g, b):
            pltpu.async_copy(
                rows_bufs[b], out_hbm.at[pl.ds(base + g * CHUNK, CHUNK)], sem_o[b]
            )

        def wait_out(b):
            pltpu.make_async_copy(
                rows_bufs[b], out_hbm.at[pl.ds(0, CHUNK)], sem_o[b]
            ).wait()

        for b in range(NBUF):
            issue_idx(b, b)
        for b in range(NBUF):
            wait_idx(b)
            issue_gather(b)

        def body(t, carry):
            g0 = t * NBUF
            for b in range(NBUF):
                wait_gather(b)
                issue_out(g0 - NBUF + b, b)
            for b in range(NBUF):
                issue_idx(g0 + b, b)
            for b in range(NBUF):
                wait_out(b)
                wait_idx(b)
                issue_gather(b)
            return carry

        lax.fori_loop(1, n_outer, body, 0)

        g0 = n_outer * NBUF
        for b in range(NBUF):
            wait_gather(b)
            issue_out(g0 - NBUF + b, b)
        for b in range(NBUF):
            wait_out(b)

    return k


def kernel(patch_indices, patch_embeddings):
    B = patch_indices.shape[0]
    V, D = patch_embeddings.shape
    idx = patch_indices.astype(jnp.int32)
    return _gather_kernel(B, V, D)(idx, patch_embeddings)


# speedup vs baseline: 4.6167x; 1.0086x over previous
import functools

import jax
import jax.numpy as jnp
from jax import lax
from jax.experimental import pallas as pl
from jax.experimental.pallas import tpu as pltpu
from jax.experimental.pallas import tpu_sc as plsc

NW = 32
NBUF = 8
CHUNK = 128


def _gather_kernel(B, V, D):
    b_per_w = B // NW
    n_chunks = b_per_w // CHUNK
    n_outer = n_chunks // NBUF
    mesh = plsc.VectorSubcoreMesh(core_axis_name="c", subcore_axis_name="s")

    scratch = (
        [pltpu.VMEM((CHUNK,), jnp.int32) for _ in range(NBUF)]
        + [pltpu.VMEM((CHUNK, D), jnp.float32) for _ in range(NBUF)]
        + [pltpu.SemaphoreType.DMA for _ in range(3 * NBUF)]
    )

    @functools.partial(
        pl.kernel,
        out_type=jax.ShapeDtypeStruct((B, D), jnp.float32),
        mesh=mesh,
        scratch_types=scratch,
        compiler_params=pltpu.CompilerParams(use_tc_tiling_on_sc=False),
    )
    def k(idx_hbm, table_hbm, out_hbm, *scratch_refs):
        idx_bufs = scratch_refs[:NBUF]
        rows_bufs = scratch_refs[NBUF : 2 * NBUF]
        sem_i = scratch_refs[2 * NBUF : 3 * NBUF]
        sem_g = scratch_refs[3 * NBUF : 4 * NBUF]
        sem_o = scratch_refs[4 * NBUF : 5 * NBUF]

        nc = 2
        wid = lax.axis_index("s") * nc + lax.axis_index("c")
        base = wid * b_per_w

        def issue_idx(g, b):
            pltpu.async_copy(
                idx_hbm.at[pl.ds(base + g * CHUNK, CHUNK)], idx_bufs[b], sem_i[b]
            )

        def wait_idx(b):
            pltpu.make_async_copy(
                idx_hbm.at[pl.ds(0, CHUNK)], idx_bufs[b], sem_i[b]
            ).wait()

        def issue_gather(b):
            pltpu.async_copy(table_hbm.at[idx_bufs[b]], rows_bufs[b], sem_g[b])

        def wait_gather(b):
            pltpu.make_async_copy(
                table_hbm.at[idx_bufs[b]], rows_bufs[b], sem_g[b]
            ).wait()

        def issue_out(g, b):
            pltpu.async_copy(
                rows_bufs[b], out_hbm.at[pl.ds(base + g * CHUNK, CHUNK)], sem_o[b]
            )

        def wait_out(b):
            pltpu.make_async_copy(
                rows_bufs[b], out_hbm.at[pl.ds(0, CHUNK)], sem_o[b]
            ).wait()

        for b in range(NBUF):
            issue_idx(b, b)
        for b in range(NBUF):
            wait_idx(b)
            issue_gather(b)

        def body(t, carry):
            g0 = t * NBUF
            for b in range(NBUF):
                wait_gather(b)
                issue_out(g0 - NBUF + b, b)
            for b in range(NBUF):
                issue_idx(g0 + b, b)
            for b in range(NBUF):
                wait_out(b)
                wait_idx(b)
                issue_gather(b)
            return carry

        lax.fori_loop(1, n_outer, body, 0)

        g0 = n_outer * NBUF
        for b in range(NBUF):
            wait_gather(b)
            issue_out(g0 - NBUF + b, b)
        for b in range(NBUF):
            wait_out(b)

    return k


def kernel(patch_indices, patch_embeddings):
    B = patch_indices.shape[0]
    V, D = patch_embeddings.shape
    idx = patch_indices.astype(jnp.int32)
    return _gather_kernel(B, V, D)(idx, patch_embeddings)
